# baseline (speedup 1.0000x reference)
import functools

import jax
import jax.numpy as jnp
from jax import lax
from jax.experimental import pallas as pl
from jax.experimental.pallas import tpu as pltpu
from jax.experimental.pallas import tpu_sc as plsc

VOCAB = 100000
EMB = 128
B = 1024
S = 512
LANES = 16
NC = 2
NS = 16
NW = NC * NS
WB = 8
WS = NW // WB
BPW = B // WB
CHUNK = S // WS
PAIR = 2
NT = BPW
NROUND = NT // PAIR


def _emb_body(idx_hbm, table_hbm, pos_hbm, out_hbm,
              idx_all, pos_all, rows_v, gsem, ssem):
    wid = lax.axis_index("s") * NC + lax.axis_index("c")
    wb = wid >> 2
    ws = wid & (WS - 1)
    b0 = wb * BPW
    sbase = ws * CHUNK
    pltpu.sync_copy(idx_hbm.at[pl.ds(b0, BPW), pl.ds(sbase, CHUNK)], idx_all)
    pltpu.sync_copy(pos_hbm.at[pl.ds(sbase, CHUNK)], pos_all)

    def gather_of(r, half, b):
        t = PAIR * r + b
        slot = half * PAIR + b
        return pltpu.make_async_copy(
            table_hbm.at[idx_all.at[t]], rows_v.at[slot], gsem.at[slot])

    def scatter_of(r, half, b):
        t = PAIR * r + b
        slot = half * PAIR + b
        return pltpu.make_async_copy(
            rows_v.at[slot],
            out_hbm.at[pl.ds((b0 + t) * S + sbase, CHUNK)],
            ssem.at[slot])

    def launch_half(r, half):
        for b in range(PAIR):
            gather_of(r, half, b).start()

    def wait_scatters(r, half):
        for b in range(PAIR):
            scatter_of(r, half, b).wait()

    def process_half(r, half):
        for b in range(PAIR):
            slot = half * PAIR + b
            gather_of(r, half, b).wait()

            @plsc.parallel_loop(0, CHUNK, unroll=4)
            def _(tt):
                for j in range(EMB // LANES):
                    sl = pl.ds(j * LANES, LANES)
                    plsc.addupdate(rows_v.at[slot, tt, sl], pos_all[tt, sl])

            scatter_of(r, half, b).start()

    def body(gg, carry):
        r0 = 2 * gg
        r1 = 2 * gg + 1

        @pl.when(gg >= 1)
        def _():
            wait_scatters(r0 - 2, 0)
            launch_half(r0, 0)
            process_half(r0 - 1, 1)
            wait_scatters(r0 - 1, 1)

        @pl.when(gg == 0)
        def _():
            launch_half(r0, 0)

        launch_half(r1, 1)
        process_half(r0, 0)
        return carry

    lax.fori_loop(0, NROUND // 2, body, 0)
    process_half(NROUND - 1, 1)
    wait_scatters(NROUND - 2, 0)
    wait_scatters(NROUND - 1, 1)


_emb = functools.partial(
    pl.kernel,
    out_type=jax.ShapeDtypeStruct((B * S, EMB), jnp.float32),
    mesh=plsc.VectorSubcoreMesh(core_axis_name="c", subcore_axis_name="s"),
    scratch_types=[
        pltpu.VMEM((NT, CHUNK), jnp.int32),
        pltpu.VMEM((CHUNK, EMB), jnp.float32),
        pltpu.VMEM((2 * PAIR, CHUNK, EMB), jnp.float32),
        pltpu.SemaphoreType.DMA((2 * PAIR,)),
        pltpu.SemaphoreType.DMA((2 * PAIR,)),
    ],
)(_emb_body)


def kernel(inputs, token_table, position_embedding):
    out = _emb(inputs.astype(jnp.int32), token_table, position_embedding[:S])
    return out.reshape(B, S, EMB)
